# baseline (device time: 99493 ns/iter reference)
import jax
import jax.numpy as jnp
from jax import lax
from jax.experimental import pallas as pl
from jax.experimental.pallas import tpu as pltpu

M_HALF = 2048
DROWS = 256
DCH = 128
NCD = DROWS // DCH
QR = 448
QCHUNKS = ((0, 32), (32, 192), (224, 112), (336, 112))
NCQ = len(QCHUNKS)
NF = NCQ // 2
FLO = ((0, 112), (112, 112))
FHI = ((224, 112), (336, 112))
QBASE = DROWS
assert QBASE + 4 * QR == M_HALF


def kernel(partial, gamma):
    _, m_global, d = partial.shape
    assert m_global // 2 == M_HALF

    def body(partial_ref, gamma_ref, out_ref, comm_ref, local_ref, stage_ref,
             zqsend, zqrecv, zdsend, zdrecv,
             dsendL, dsendR, drecvL, drecvR,
             fsendL, fsendR, frecvL, frecvR, copy_sem, osem):
        my_x = lax.axis_index("x")
        my_y = lax.axis_index("y")
        my_z = lax.axis_index("z")
        partner = (my_x, my_y, 1 - my_z)

        p = jnp.where(my_x == 0, my_y, 3 - my_y)

        def pos_xy(q):
            q = q % 4
            return q // 2, ((q + 1) % 4) // 2

        lx, ly = pos_xy(p - 1)
        rx, ry = pos_xy(p + 1)
        left = (lx, ly, my_z)
        right = (rx, ry, my_z)

        def rows_q(qi, off):
            return QBASE + (qi % 4) * QR + off

        local_cp = pltpu.make_async_copy(
            partial_ref.at[0, pl.ds(my_z * M_HALF, M_HALF), :],
            local_ref,
            copy_sem,
        )
        local_cp.start()

        barrier_sem = pltpu.get_barrier_semaphore()
        for nbr in (partner, left, right):
            pl.semaphore_signal(
                barrier_sem, inc=1,
                device_id=nbr, device_id_type=pl.DeviceIdType.MESH,
            )
        pl.semaphore_wait(barrier_sem, 3)

        zq_send = [
            pltpu.make_async_remote_copy(
                src_ref=partial_ref.at[
                    0, pl.ds((1 - my_z) * M_HALF + rows_q(p, off), sz), :
                ],
                dst_ref=comm_ref.at[pl.ds(rows_q(p, off), sz), :],
                send_sem=zqsend.at[c],
                recv_sem=zqrecv.at[c],
                device_id=partner,
                device_id_type=pl.DeviceIdType.MESH,
            )
            for c, (off, sz) in enumerate(QCHUNKS)
        ]
        zd_send = [
            pltpu.make_async_remote_copy(
                src_ref=partial_ref.at[
                    0, pl.ds((1 - my_z) * M_HALF + c * DCH, DCH), :
                ],
                dst_ref=comm_ref.at[pl.ds(c * DCH, DCH), :],
                send_sem=zdsend.at[c],
                recv_sem=zdrecv.at[c],
                device_id=partner,
                device_id_type=pl.DeviceIdType.MESH,
            )
            for c in range(NCD)
        ]
        d_sendR = [
            pltpu.make_async_remote_copy(
                src_ref=comm_ref.at[pl.ds(rows_q(p, off), sz), :],
                dst_ref=comm_ref.at[pl.ds(rows_q(p, off), sz), :],
                send_sem=dsendR.at[c],
                recv_sem=drecvL.at[c],
                device_id=right,
                device_id_type=pl.DeviceIdType.MESH,
            )
            for c, (off, sz) in enumerate(QCHUNKS)
        ]
        d_sendL = [
            pltpu.make_async_remote_copy(
                src_ref=comm_ref.at[pl.ds(rows_q(p, off), sz), :],
                dst_ref=comm_ref.at[pl.ds(rows_q(p, off), sz), :],
                send_sem=dsendL.at[c],
                recv_sem=drecvR.at[c],
                device_id=left,
                device_id_type=pl.DeviceIdType.MESH,
            )
            for c, (off, sz) in enumerate(QCHUNKS)
        ]
        f_sendR = [
            pltpu.make_async_remote_copy(
                src_ref=comm_ref.at[pl.ds(rows_q(p - 1, off), sz), :],
                dst_ref=comm_ref.at[pl.ds(rows_q(p - 1, off), sz), :],
                send_sem=fsendR.at[c],
                recv_sem=frecvL.at[c],
                device_id=right,
                device_id_type=pl.DeviceIdType.MESH,
            )
            for c, (off, sz) in enumerate(FHI)
        ]
        f_sendL = [
            pltpu.make_async_remote_copy(
                src_ref=comm_ref.at[pl.ds(rows_q(p + 1, off), sz), :],
                dst_ref=comm_ref.at[pl.ds(rows_q(p + 1, off), sz), :],
                send_sem=fsendL.at[c],
                recv_sem=frecvR.at[c],
                device_id=left,
                device_id_type=pl.DeviceIdType.MESH,
            )
            for c, (off, sz) in enumerate(FLO)
        ]

        def recv_only(r0, n, sem):
            return pltpu.make_async_remote_copy(
                src_ref=comm_ref.at[pl.ds(r0, n), :],
                dst_ref=comm_ref.at[pl.ds(r0, n), :],
                send_sem=zqsend.at[0],
                recv_sem=sem,
                device_id=partner,
                device_id_type=pl.DeviceIdType.MESH,
            )

        zq_recv = [recv_only(rows_q(p, off), sz, zqrecv.at[c])
                   for c, (off, sz) in enumerate(QCHUNKS)]
        zd_recv = [recv_only(c * DCH, DCH, zdrecv.at[c]) for c in range(NCD)]
        dL_recv = [recv_only(rows_q(p - 1, off), sz, drecvL.at[c])
                   for c, (off, sz) in enumerate(QCHUNKS)]
        dR_recv = [recv_only(rows_q(p + 1, off), sz, drecvR.at[c])
                   for c, (off, sz) in enumerate(QCHUNKS)]
        fL_recv = [recv_only(rows_q(p + 2, off), sz, frecvL.at[c])
                   for c, (off, sz) in enumerate(FHI)]
        fR_recv = [recv_only(rows_q(p + 2, off), sz, frecvR.at[c])
                   for c, (off, sz) in enumerate(FLO)]

        out_cps = []

        def compute_rows(r0, n):
            y = local_ref[pl.ds(r0, n), :] + comm_ref[pl.ds(r0, n), :]
            rms = jnp.sqrt(jnp.mean(y * y, axis=-1, keepdims=True) + 1e-6)
            stage_ref[pl.ds(r0, n), :] = (y / rms) * gamma_ref[:][None, :]
            cp = pltpu.make_async_copy(
                stage_ref.at[pl.ds(r0, n), :],
                out_ref.at[pl.ds(r0, n), :],
                osem.at[len(out_cps)],
            )
            cp.start()
            out_cps.append(cp)

        for c in range(NCQ):
            zq_send[c].start()
        for c in range(NCD):
            zd_send[c].start()
        local_cp.wait()

        for c, (off, sz) in enumerate(QCHUNKS):
            zq_recv[c].wait_recv()
            d_sendR[c].start()
            d_sendL[c].start()
            compute_rows(rows_q(p, off), sz)

        for c, (off, sz) in enumerate(QCHUNKS):
            dL_recv[c].wait_recv()
            if c >= NF:
                f_sendR[c - NF].start()
            compute_rows(rows_q(p - 1, off), sz)
            dR_recv[c].wait_recv()
            if c == 1:
                f_sendL[0].start()
                f_sendL[1].start()
            compute_rows(rows_q(p + 1, off), sz)

        for c in range(NCD):
            zd_recv[c].wait_recv()
            compute_rows(c * DCH, DCH)

        for c in range(NF):
            fR_recv[c].wait_recv()
            compute_rows(rows_q(p + 2, FLO[c][0]), FLO[c][1])
            fL_recv[c].wait_recv()
            compute_rows(rows_q(p + 2, FHI[c][0]), FHI[c][1])

        for c in range(NCQ):
            zq_send[c].wait_send()
            d_sendR[c].wait_send()
            d_sendL[c].wait_send()
        for c in range(NCD):
            zd_send[c].wait_send()
        for c in range(NF):
            f_sendR[c].wait_send()
            f_sendL[c].wait_send()
        for cp in out_cps:
            cp.wait()

    n_outcp = NCQ + 2 * NCQ + NCD + 2 * NF
    return pl.pallas_call(
        body,
        out_shape=jax.ShapeDtypeStruct((M_HALF, d), jnp.float32),
        in_specs=[
            pl.BlockSpec(memory_space=pl.ANY),
            pl.BlockSpec(memory_space=pltpu.VMEM),
        ],
        out_specs=pl.BlockSpec(memory_space=pl.ANY),
        scratch_shapes=[
            pltpu.VMEM((M_HALF, d), jnp.float32),
            pltpu.VMEM((M_HALF, d), jnp.float32),
            pltpu.VMEM((M_HALF, d), jnp.float32),
            pltpu.SemaphoreType.DMA((NCQ,)),
            pltpu.SemaphoreType.DMA((NCQ,)),
            pltpu.SemaphoreType.DMA((NCD,)),
            pltpu.SemaphoreType.DMA((NCD,)),
            pltpu.SemaphoreType.DMA((NCQ,)),
            pltpu.SemaphoreType.DMA((NCQ,)),
            pltpu.SemaphoreType.DMA((NCQ,)),
            pltpu.SemaphoreType.DMA((NCQ,)),
            pltpu.SemaphoreType.DMA((NF,)),
            pltpu.SemaphoreType.DMA((NF,)),
            pltpu.SemaphoreType.DMA((NF,)),
            pltpu.SemaphoreType.DMA((NF,)),
            pltpu.SemaphoreType.DMA,
            pltpu.SemaphoreType.DMA((n_outcp,)),
        ],
        compiler_params=pltpu.CompilerParams(
            collective_id=0,
            vmem_limit_bytes=56 * 1024 * 1024,
        ),
    )(partial, gamma)


# device time: 90859 ns/iter; 1.0950x vs baseline; 1.0950x over previous
import jax
import jax.numpy as jnp
from jax import lax
from jax.experimental import pallas as pl
from jax.experimental.pallas import tpu as pltpu

M_HALF = 2048
DROWS = 256
DCH = 128
NCD = DROWS // DCH
QR = 448
QCHUNKS = ((0, 112), (112, 112), (224, 112), (336, 112))
NCQ = len(QCHUNKS)
NF = NCQ // 2
FLO = ((0, 112), (112, 112))
FHI = ((224, 112), (336, 112))
QBASE = DROWS
assert QBASE + 4 * QR == M_HALF


def kernel(partial, gamma):
    _, m_global, d = partial.shape
    assert m_global // 2 == M_HALF

    def body(partial_ref, gamma_ref, out_ref, comm_ref, local_ref, stage_ref,
             zqsend, zqrecv, zdsend, zdrecv,
             dsendL, dsendR, drecvL, drecvR,
             fsendL, fsendR, frecvL, frecvR, copy_sem, osem):
        my_x = lax.axis_index("x")
        my_y = lax.axis_index("y")
        my_z = lax.axis_index("z")
        partner = (my_x, my_y, 1 - my_z)

        p = jnp.where(my_x == 0, my_y, 3 - my_y)

        def pos_xy(q):
            q = q % 4
            return q // 2, ((q + 1) % 4) // 2

        lx, ly = pos_xy(p - 1)
        rx, ry = pos_xy(p + 1)
        left = (lx, ly, my_z)
        right = (rx, ry, my_z)

        def rows_q(qi, off):
            return QBASE + (qi % 4) * QR + off

        local_cp = pltpu.make_async_copy(
            partial_ref.at[0, pl.ds(my_z * M_HALF, M_HALF), :],
            local_ref,
            copy_sem,
        )
        local_cp.start()

        barrier_sem = pltpu.get_barrier_semaphore()
        for nbr in (partner, left, right):
            pl.semaphore_signal(
                barrier_sem, inc=1,
                device_id=nbr, device_id_type=pl.DeviceIdType.MESH,
            )
        pl.semaphore_wait(barrier_sem, 3)

        zq_send = [
            pltpu.make_async_remote_copy(
                src_ref=partial_ref.at[
                    0, pl.ds((1 - my_z) * M_HALF + rows_q(p, off), sz), :
                ],
                dst_ref=comm_ref.at[pl.ds(rows_q(p, off), sz), :],
                send_sem=zqsend.at[c],
                recv_sem=zqrecv.at[c],
                device_id=partner,
                device_id_type=pl.DeviceIdType.MESH,
            )
            for c, (off, sz) in enumerate(QCHUNKS)
        ]
        zd_send = [
            pltpu.make_async_remote_copy(
                src_ref=partial_ref.at[
                    0, pl.ds((1 - my_z) * M_HALF + c * DCH, DCH), :
                ],
                dst_ref=comm_ref.at[pl.ds(c * DCH, DCH), :],
                send_sem=zdsend.at[c],
                recv_sem=zdrecv.at[c],
                device_id=partner,
                device_id_type=pl.DeviceIdType.MESH,
            )
            for c in range(NCD)
        ]
        d_sendR = [
            pltpu.make_async_remote_copy(
                src_ref=comm_ref.at[pl.ds(rows_q(p, off), sz), :],
                dst_ref=comm_ref.at[pl.ds(rows_q(p, off), sz), :],
                send_sem=dsendR.at[c],
                recv_sem=drecvL.at[c],
                device_id=right,
                device_id_type=pl.DeviceIdType.MESH,
            )
            for c, (off, sz) in enumerate(QCHUNKS)
        ]
        d_sendL = [
            pltpu.make_async_remote_copy(
                src_ref=comm_ref.at[pl.ds(rows_q(p, off), sz), :],
                dst_ref=comm_ref.at[pl.ds(rows_q(p, off), sz), :],
                send_sem=dsendL.at[c],
                recv_sem=drecvR.at[c],
                device_id=left,
                device_id_type=pl.DeviceIdType.MESH,
            )
            for c, (off, sz) in enumerate(QCHUNKS)
        ]
        f_sendR = [
            pltpu.make_async_remote_copy(
                src_ref=comm_ref.at[pl.ds(rows_q(p - 1, off), sz), :],
                dst_ref=comm_ref.at[pl.ds(rows_q(p - 1, off), sz), :],
                send_sem=fsendR.at[c],
                recv_sem=frecvL.at[c],
                device_id=right,
                device_id_type=pl.DeviceIdType.MESH,
            )
            for c, (off, sz) in enumerate(FHI)
        ]
        f_sendL = [
            pltpu.make_async_remote_copy(
                src_ref=comm_ref.at[pl.ds(rows_q(p + 1, off), sz), :],
                dst_ref=comm_ref.at[pl.ds(rows_q(p + 1, off), sz), :],
                send_sem=fsendL.at[c],
                recv_sem=frecvR.at[c],
                device_id=left,
                device_id_type=pl.DeviceIdType.MESH,
            )
            for c, (off, sz) in enumerate(FLO)
        ]

        def recv_only(r0, n, sem):
            return pltpu.make_async_remote_copy(
                src_ref=comm_ref.at[pl.ds(r0, n), :],
                dst_ref=comm_ref.at[pl.ds(r0, n), :],
                send_sem=zqsend.at[0],
                recv_sem=sem,
                device_id=partner,
                device_id_type=pl.DeviceIdType.MESH,
            )

        zq_recv = [recv_only(rows_q(p, off), sz, zqrecv.at[c])
                   for c, (off, sz) in enumerate(QCHUNKS)]
        zd_recv = [recv_only(c * DCH, DCH, zdrecv.at[c]) for c in range(NCD)]
        dL_recv = [recv_only(rows_q(p - 1, off), sz, drecvL.at[c])
                   for c, (off, sz) in enumerate(QCHUNKS)]
        dR_recv = [recv_only(rows_q(p + 1, off), sz, drecvR.at[c])
                   for c, (off, sz) in enumerate(QCHUNKS)]
        fL_recv = [recv_only(rows_q(p + 2, off), sz, frecvL.at[c])
                   for c, (off, sz) in enumerate(FHI)]
        fR_recv = [recv_only(rows_q(p + 2, off), sz, frecvR.at[c])
                   for c, (off, sz) in enumerate(FLO)]

        out_cps = []

        def compute_rows(r0, n):
            y = local_ref[pl.ds(r0, n), :] + comm_ref[pl.ds(r0, n), :]
            rms = jnp.sqrt(jnp.mean(y * y, axis=-1, keepdims=True) + 1e-6)
            stage_ref[pl.ds(r0, n), :] = (y / rms) * gamma_ref[:][None, :]
            cp = pltpu.make_async_copy(
                stage_ref.at[pl.ds(r0, n), :],
                out_ref.at[pl.ds(r0, n), :],
                osem.at[len(out_cps)],
            )
            cp.start()
            out_cps.append(cp)

        for c in range(NCQ):
            zq_send[c].start()
        for c in range(NCD):
            zd_send[c].start()
        local_cp.wait()

        for c, (off, sz) in enumerate(QCHUNKS):
            zq_recv[c].wait_recv()
            d_sendR[c].start()
            d_sendL[c].start()
            compute_rows(rows_q(p, off), sz)

        for c, (off, sz) in enumerate(QCHUNKS):
            dL_recv[c].wait_recv()
            if c >= NF:
                f_sendR[c - NF].start()
            compute_rows(rows_q(p - 1, off), sz)
            dR_recv[c].wait_recv()
            if c < NF:
                f_sendL[c].start()
            compute_rows(rows_q(p + 1, off), sz)

        for c in range(NCD):
            zd_recv[c].wait_recv()
            compute_rows(c * DCH, DCH)

        for c in range(NF):
            fR_recv[c].wait_recv()
            compute_rows(rows_q(p + 2, FLO[c][0]), FLO[c][1])
            fL_recv[c].wait_recv()
            compute_rows(rows_q(p + 2, FHI[c][0]), FHI[c][1])

        for c in range(NCQ):
            zq_send[c].wait_send()
            d_sendR[c].wait_send()
            d_sendL[c].wait_send()
        for c in range(NCD):
            zd_send[c].wait_send()
        for c in range(NF):
            f_sendR[c].wait_send()
            f_sendL[c].wait_send()
        for cp in out_cps:
            cp.wait()

    n_outcp = NCQ + 2 * NCQ + NCD + 2 * NF
    return pl.pallas_call(
        body,
        out_shape=jax.ShapeDtypeStruct((M_HALF, d), jnp.float32),
        in_specs=[
            pl.BlockSpec(memory_space=pl.ANY),
            pl.BlockSpec(memory_space=pltpu.VMEM),
        ],
        out_specs=pl.BlockSpec(memory_space=pl.ANY),
        scratch_shapes=[
            pltpu.VMEM((M_HALF, d), jnp.float32),
            pltpu.VMEM((M_HALF, d), jnp.float32),
            pltpu.VMEM((M_HALF, d), jnp.float32),
            pltpu.SemaphoreType.DMA((NCQ,)),
            pltpu.SemaphoreType.DMA((NCQ,)),
            pltpu.SemaphoreType.DMA((NCD,)),
            pltpu.SemaphoreType.DMA((NCD,)),
            pltpu.SemaphoreType.DMA((NCQ,)),
            pltpu.SemaphoreType.DMA((NCQ,)),
            pltpu.SemaphoreType.DMA((NCQ,)),
            pltpu.SemaphoreType.DMA((NCQ,)),
            pltpu.SemaphoreType.DMA((NF,)),
            pltpu.SemaphoreType.DMA((NF,)),
            pltpu.SemaphoreType.DMA((NF,)),
            pltpu.SemaphoreType.DMA((NF,)),
            pltpu.SemaphoreType.DMA,
            pltpu.SemaphoreType.DMA((n_outcp,)),
        ],
        compiler_params=pltpu.CompilerParams(
            collective_id=0,
            vmem_limit_bytes=56 * 1024 * 1024,
        ),
    )(partial, gamma)
